# baseline (speedup 1.0000x reference)
import functools

import jax
import jax.numpy as jnp
from jax import lax
from jax.experimental import pallas as pl
from jax.experimental.pallas import tpu as pltpu
from jax.experimental.pallas import tpu_sc as plsc

LAT = 16
BATCH = 16384
NC = 2
NS = 16
NW = NC * NS
PER_W = BATCH // NW
NROW = 1000000
RSPAN = 30720
TAILR = NW * RSPAN
CW = 512
NCHK = RSPAN // CW
IDXCH = 2048
NSLOT = 128
SPILL = 112


def _iota():
    return lax.iota(jnp.int32, LAT)


def _stage_body(uidx_hbm, iidx_hbm, utab_hbm, itab_hbm,
                ustage_hbm, istage_hbm,
                idxb_u, idxb_i, uel_v, url_v, iel_v, irl_v,
                cb_v, slot_v, semc0, semc1, semh):
    c = lax.axis_index("c")
    s = lax.axis_index("s")
    wid = s * NC + c
    lo = wid * RSPAN
    hi = lo + RSPAN
    iota = _iota()

    def scan_super(ss, carry):
        cu, ci = carry
        pltpu.sync_copy(uidx_hbm.at[pl.ds(ss * IDXCH, IDXCH)], idxb_u)
        pltpu.sync_copy(iidx_hbm.at[pl.ds(ss * IDXCH, IDXCH)], idxb_i)

        def scan_vreg(j, carry2):
            cu2, ci2 = carry2
            el = ss * IDXCH + j * LAT + iota
            u = idxb_u[pl.ds(j * LAT, LAT)]
            mu = (u >= lo) & (u < hi)
            posu = cu2 + plsc.cumsum(mu.astype(jnp.int32)) - 1
            plsc.store_scatter(uel_v, [posu], el, mask=mu)
            plsc.store_scatter(url_v, [posu], u, mask=mu)
            cu2 = cu2 + plsc.all_reduce_population_count(mu)[0]
            it = idxb_i[pl.ds(j * LAT, LAT)]
            mi = (it >= lo) & (it < hi)
            posi = ci2 + plsc.cumsum(mi.astype(jnp.int32)) - 1
            plsc.store_scatter(iel_v, [posi], el, mask=mi)
            plsc.store_scatter(irl_v, [posi], it, mask=mi)
            ci2 = ci2 + plsc.all_reduce_population_count(mi)[0]
            return (cu2, ci2)

        return lax.fori_loop(0, IDXCH // LAT, scan_vreg, (cu, ci))

    cnt_u, cnt_i = lax.fori_loop(0, BATCH // IDXCH, scan_super, (0, 0))

    sems = [semc0, semc1]

    def fire(t, par):
        r0 = pl.multiple_of(lo + t * CW, 128)
        pltpu.async_copy(utab_hbm.at[:, pl.ds(r0, CW)], cb_v.at[par, 0],
                         sems[par])
        pltpu.async_copy(itab_hbm.at[:, pl.ds(r0, CW)], cb_v.at[par, 1],
                         sems[par])

    def drainc(par):
        pltpu.make_async_copy(utab_hbm.at[:, pl.ds(0, CW)], cb_v.at[par, 0],
                              sems[par]).wait()
        pltpu.make_async_copy(utab_hbm.at[:, pl.ds(0, CW)], cb_v.at[par, 1],
                              sems[par]).wait()

    def unit_drain():
        pltpu.make_async_copy(slot_v.at[0], ustage_hbm.at[pl.ds(0, LAT)],
                              semh).wait()

    def serve_tab(par, tab, el_list, r_list, cnt, r0, stage_hbm, state):
        slotc, pend = state
        nb = (cnt + LAT - 1) // LAT

        def blk(b, st):
            slotc2, pend2 = st
            elv = el_list[pl.ds(b * LAT, LAT)]
            rv = r_list[pl.ds(b * LAT, LAT)]
            valid = (b * LAT + iota) < cnt
            m = (rv >= r0) & (rv < r0 + CW) & valid
            pop = plsc.all_reduce_population_count(m)[0]
            mi = m.astype(jnp.int32)
            rankv = plsc.cumsum(mi) - 1
            colv = rv - r0

            @pl.when(pop > 0)
            def _():
                for h in range(LAT):
                    @pl.when(mi[h] != 0)
                    def _():
                        vals = plsc.load_gather(
                            cb_v, [iota * 0 + par, iota * 0 + tab,
                                   iota, iota * 0 + colv[h]])
                        sl = (slotc2 + rankv[h]) % NSLOT
                        slot_v[sl, :] = vals
                        pltpu.async_copy(
                            slot_v.at[sl],
                            stage_hbm.at[pl.ds(elv[h] * LAT, LAT)], semh)

            slotc2 = slotc2 + pop
            pend2 = pend2 + pop
            need = pend2 > SPILL

            @pl.when(need)
            def _():
                def dr(q, cc):
                    unit_drain()
                    return cc

                lax.fori_loop(0, pend2, dr, 0)

            pend2 = jnp.where(need, 0, pend2)
            return (slotc2, pend2)

        return lax.fori_loop(0, nb, blk, (slotc, pend))

    def serve(t, par, state):
        r0 = lo + t * CW
        state = serve_tab(par, 0, uel_v, url_v, cnt_u, r0, ustage_hbm, state)
        state = serve_tab(par, 1, iel_v, irl_v, cnt_i, r0, istage_hbm, state)
        return state

    fire(0, 0)

    def pair(k, state):
        g = k * 2
        fire(g + 1, 1)
        drainc(0)
        state = serve(g, 0, state)
        fire(g + 2, 0)
        drainc(1)
        state = serve(g + 1, 1, state)
        return state

    state = lax.fori_loop(0, NCHK // 2 - 1, pair, (0, 0))
    g = NCHK - 2
    fire(g + 1, 1)
    drainc(0)
    state = serve(g, 0, state)
    drainc(1)
    state = serve(g + 1, 1, state)

    _, pend = state

    def fin(q, carry):
        unit_drain()
        return carry

    lax.fori_loop(0, pend, fin, 0)


def _serve_tab_fixup(tab_hbm, idx_v, buf_v, win_v, iota):
    def blk(b, carry):
        v = idx_v[pl.ds(b * LAT, LAT)]
        m = v >= TAILR
        pop = plsc.all_reduce_population_count(m)[0]
        mi = m.astype(jnp.int32)

        @pl.when(pop > 0)
        def _():
            for h in range(LAT):
                @pl.when(mi[h] != 0)
                def _():
                    r = v[h]
                    ws = pl.multiple_of((r >> 7) * 128, 128)
                    pltpu.sync_copy(tab_hbm.at[:, pl.ds(ws, 128)], win_v)
                    vals = plsc.load_gather(
                        win_v, [iota, iota * 0 + (r & 127)])
                    word = (b * LAT + h) * LAT
                    cs = pl.multiple_of(word & 127, LAT)
                    buf_v[word >> 7, pl.ds(cs, LAT)] = vals

        return carry

    lax.fori_loop(0, PER_W // LAT, blk, 0)


def _compute_body(ustage_hbm, istage_hbm, wb_hbm, uidx_hbm, iidx_hbm,
                  utab_hbm, itab_hbm, out_hbm,
                  uv, iv, uidx_v, iidx_v, win_v, wb_v, out_v):
    c = lax.axis_index("c")
    s = lax.axis_index("s")
    wid = s * NC + c
    iota = _iota()
    rows = PER_W * LAT // 128

    pltpu.sync_copy(ustage_hbm.at[pl.ds(wid * rows, rows)], uv)
    pltpu.sync_copy(istage_hbm.at[pl.ds(wid * rows, rows)], iv)
    pltpu.sync_copy(uidx_hbm.at[wid], uidx_v.at[pl.ds(0, PER_W)])
    pltpu.sync_copy(iidx_hbm.at[wid], iidx_v.at[pl.ds(0, PER_W)])
    pltpu.sync_copy(wb_hbm, wb_v)

    _serve_tab_fixup(utab_hbm, uidx_v, uv, win_v, iota)
    _serve_tab_fixup(itab_hbm, iidx_v, iv, win_v, iota)

    wvs = [plsc.load_gather(wb_v, [iota * 0 + d, iota]) for d in range(LAT)]
    bvec = wb_v[LAT, :]

    def body(j, carry):
        wordb = (j * LAT + iota) * LAT
        acc = bvec
        for d in range(LAT):
            wd = wordb + d
            u = plsc.load_gather(uv, [wd >> 7, wd & 127])
            it = plsc.load_gather(iv, [wd >> 7, wd & 127])
            acc = acc + u * it * wvs[d]
        out_v[pl.ds(j * LAT, LAT)] = acc
        return carry

    lax.fori_loop(0, PER_W // LAT, body, 0)

    pltpu.sync_copy(out_v, out_hbm.at[pl.ds(wid * PER_W, PER_W)])


_mesh = plsc.VectorSubcoreMesh(core_axis_name="c", subcore_axis_name="s")
_params = pltpu.CompilerParams(
    needs_layout_passes=False, use_tc_tiling_on_sc=True)

_stage = functools.partial(
    pl.kernel,
    out_type=(jax.ShapeDtypeStruct((BATCH * LAT,), jnp.float32),
              jax.ShapeDtypeStruct((BATCH * LAT,), jnp.float32)),
    mesh=_mesh,
    scratch_types=[
        pltpu.VMEM((IDXCH,), jnp.int32),
        pltpu.VMEM((IDXCH,), jnp.int32),
        pltpu.VMEM((BATCH,), jnp.int32),
        pltpu.VMEM((BATCH,), jnp.int32),
        pltpu.VMEM((BATCH,), jnp.int32),
        pltpu.VMEM((BATCH,), jnp.int32),
        pltpu.VMEM((2, 2, LAT, CW), jnp.float32),
        pltpu.VMEM((NSLOT, LAT), jnp.float32),
        pltpu.SemaphoreType.DMA,
        pltpu.SemaphoreType.DMA,
        pltpu.SemaphoreType.DMA,
    ],
    compiler_params=_params,
)(_stage_body)

_compute = functools.partial(
    pl.kernel,
    out_type=jax.ShapeDtypeStruct((BATCH,), jnp.float32),
    mesh=_mesh,
    scratch_types=[
        pltpu.VMEM((PER_W * LAT // 128, 128), jnp.float32),
        pltpu.VMEM((PER_W * LAT // 128, 128), jnp.float32),
        pltpu.VMEM((PER_W + 8,), jnp.int32),
        pltpu.VMEM((PER_W + 8,), jnp.int32),
        pltpu.VMEM((LAT, 128), jnp.float32),
        pltpu.VMEM((LAT + 1, LAT), jnp.float32),
        pltpu.VMEM((PER_W,), jnp.float32),
    ],
    compiler_params=_params,
)(_compute_body)


@jax.jit
def kernel(user_indices, item_indices, user_table, item_table, W, b):
    uflat = user_indices.astype(jnp.int32)
    iflat = item_indices.astype(jnp.int32)
    ust, ist = _stage(uflat, iflat, user_table.T, item_table.T)
    wb = jnp.concatenate([W.reshape(LAT), b.reshape(1)])
    wb = jnp.broadcast_to(wb[:, None], (LAT + 1, LAT))
    out = _compute(ust.reshape(-1, 128), ist.reshape(-1, 128), wb,
                   uflat.reshape(NW, PER_W), iflat.reshape(NW, PER_W),
                   user_table.T, item_table.T)
    return out.reshape(BATCH, 1)


# speedup vs baseline: 3.0533x; 3.0533x over previous
import functools

import jax
import jax.numpy as jnp
from jax import lax
from jax.experimental import pallas as pl
from jax.experimental.pallas import tpu as pltpu
from jax.experimental.pallas import tpu_sc as plsc

LAT = 16
BATCH = 16384
NC = 2
NS = 16
NW = NC * NS
PER_W = BATCH // NW
GRP = 8
NGRP = PER_W // GRP


def _gmf_body(uidx_hbm, iidx_hbm, utab_hbm, itab_hbm, wb_hbm, out_hbm,
              uidx_v, iidx_v, win_v, wb_v, out_v, sem0, sem1):
    c = lax.axis_index("c")
    s = lax.axis_index("s")
    wid = s * NC + c

    pltpu.sync_copy(uidx_hbm.at[wid], uidx_v.at[pl.ds(0, PER_W)])
    pltpu.sync_copy(iidx_hbm.at[wid], iidx_v.at[pl.ds(0, PER_W)])
    pltpu.sync_copy(wb_hbm, wb_v)

    sems = [sem0, sem1]

    def fire(g, buf):
        base = g * GRP
        uwv = (uidx_v[pl.ds(base, LAT)] >> 7) * 128
        iwv = (iidx_v[pl.ds(base, LAT)] >> 7) * 128
        for e in range(GRP):
            us = pl.multiple_of(uwv[e], 128)
            is_ = pl.multiple_of(iwv[e], 128)
            pltpu.async_copy(
                utab_hbm.at[:, pl.ds(us, 128)], win_v.at[buf, e], sems[buf])
            pltpu.async_copy(
                itab_hbm.at[:, pl.ds(is_, 128)], win_v.at[buf, GRP + e],
                sems[buf])

    def drain(buf):
        pltpu.make_async_copy(
            utab_hbm.at[:, pl.ds(0, 2 * GRP * 128)],
            win_v.at[buf].reshape(LAT, 2 * GRP * 128), sems[buf]).wait()

    lane = lax.iota(jnp.int32, LAT)
    elane = lane % GRP
    dvecs = [(lane // GRP) * 8 + dd for dd in range(GRP)]
    zeros = lane * 0
    lowm = lane < GRP
    highm = lane >= GRP
    wvs = [plsc.load_gather(wb_v, [dvecs[dd], zeros]) for dd in range(GRP)]
    bvec = wb_v[LAT, :]

    def compute(g, sbuf):
        buf = zeros + sbuf
        base = g * GRP
        epos = base + elane
        ucol = plsc.load_gather(uidx_v, [epos]) & 127
        icol = plsc.load_gather(iidx_v, [epos]) & 127
        acc = lax.full((LAT,), 0.0, jnp.float32)
        for dd in range(GRP):
            u = plsc.load_gather(win_v, [buf, elane, dvecs[dd], ucol])
            it = plsc.load_gather(
                win_v, [buf, elane + GRP, dvecs[dd], icol])
            acc = acc + u * it * wvs[dd]
        plsc.store_scatter(out_v, [epos], acc + bvec, mask=lowm)
        plsc.addupdate_scatter(out_v, [epos], acc, mask=highm)
        return ()

    fire(0, 0)

    def body(k, carry):
        g = k * 2
        fire(g + 1, 1)
        drain(0)
        compute(g, 0)
        fire(g + 2, 0)
        drain(1)
        compute(g + 1, 1)
        return carry

    lax.fori_loop(0, NGRP // 2 - 1, body, 0)
    g = NGRP - 2
    fire(g + 1, 1)
    drain(0)
    compute(g, 0)
    drain(1)
    compute(g + 1, 1)

    pltpu.sync_copy(out_v, out_hbm.at[pl.ds(wid * PER_W, PER_W)])


_gmf = functools.partial(
    pl.kernel,
    out_type=jax.ShapeDtypeStruct((BATCH,), jnp.float32),
    mesh=plsc.VectorSubcoreMesh(core_axis_name="c", subcore_axis_name="s"),
    scratch_types=[
        pltpu.VMEM((PER_W + 8,), jnp.int32),
        pltpu.VMEM((PER_W + 8,), jnp.int32),
        pltpu.VMEM((2, 2 * GRP, LAT, 128), jnp.float32),
        pltpu.VMEM((LAT + 1, LAT), jnp.float32),
        pltpu.VMEM((PER_W,), jnp.float32),
        pltpu.SemaphoreType.DMA,
        pltpu.SemaphoreType.DMA,
    ],
    compiler_params=pltpu.CompilerParams(
        needs_layout_passes=False, use_tc_tiling_on_sc=True),
)(_gmf_body)


@jax.jit
def kernel(user_indices, item_indices, user_table, item_table, W, b):
    uidx = user_indices.astype(jnp.int32).reshape(NW, PER_W)
    iidx = item_indices.astype(jnp.int32).reshape(NW, PER_W)
    wb = jnp.concatenate([W.reshape(LAT), b.reshape(1)])
    wb = jnp.broadcast_to(wb[:, None], (LAT + 1, LAT))
    out = _gmf(uidx, iidx, user_table.T, item_table.T, wb)
    return out.reshape(BATCH, 1)
